# Initial kernel scaffold; baseline (speedup 1.0000x reference)
import jax
import jax.numpy as jnp
from jax.experimental import pallas as pl


def kernel(x, gate_weight, experts):
    raise NotImplementedError("write your pallas kernel here")


# speedup vs baseline: 1.4800x; 1.4800x over previous
import dataclasses
import functools

import jax
import jax.numpy as jnp
from jax import lax
from jax.experimental import pallas as pl
from jax.experimental.pallas import tpu as pltpu
from jax.experimental.pallas import tpu_sc as plsc

B, N, D, E = 4, 4096, 2048, 8
M = N // E
W = B * E
NZ = 2048
GW = 16

@functools.cache
def _mesh():
    return plsc.VectorSubcoreMesh(core_axis_name="c", subcore_axis_name="s")


def _sc_params():
    cp = pltpu.CompilerParams()
    if "needs_layout_passes" in pltpu.CompilerParams.__dataclass_fields__:
        cp = dataclasses.replace(cp, needs_layout_passes=False)
    return cp


def _k1_body(x_ref, gw_ref, gates_ref, meta_ref, logits_ref):
    nb = pl.program_id(1)
    xb = x_ref[0]
    lg = lax.dot_general(gw_ref[...], xb, (((0,), (1,)), ((), ())),
                         preferred_element_type=jnp.float32)
    logits_ref[:, pl.ds(nb * (N // 8), N // 8)] = lg

    @pl.when(nb == 7)
    def _():
        t = logits_ref[...]
        t = jnp.log(jnp.clip(t, 1e-6, None))
        for _ in range(8):
            m1 = jnp.max(t, axis=1, keepdims=True)
            t = t - (jnp.log(jnp.sum(jnp.exp(t - m1), axis=1,
                                     keepdims=True)) + m1)
            m0 = jnp.max(t, axis=0, keepdims=True)
            t = t - (jnp.log(jnp.sum(jnp.exp(t - m0), axis=0,
                                     keepdims=True)) + m0)
        g = jnp.exp(t)
        gates_ref[0] = g
        gi = lax.bitcast_convert_type(g, jnp.int32)

        def bs(_, lohi):
            lo, hi = lohi
            mid = lo + (hi - lo + 1) // 2
            cnt = jnp.sum((gi >= mid).astype(jnp.int32), axis=1,
                          keepdims=True)
            ok = cnt >= M
            return jnp.where(ok, mid, lo), jnp.where(ok, hi, mid - 1)

        lo, _hi = lax.fori_loop(
            0, 31, bs,
            (jnp.zeros((E, 1), jnp.int32),
             jnp.full((E, 1), 0x7F800000, jnp.int32)))
        cnt_gt = jnp.sum((gi > lo).astype(jnp.int32), axis=1, keepdims=True)
        need = M - cnt_gt
        meta_ref[0] = jnp.concatenate(
            [lo, need, jnp.zeros((E, 126), jnp.int32)], axis=1)


def _route_tc(x, gw):
    return pl.pallas_call(
        _k1_body,
        grid=(B, 8),
        in_specs=[
            pl.BlockSpec((1, N // 8, D), lambda b, nb: (b, nb, 0)),
            pl.BlockSpec((D, E), lambda b, nb: (0, 0)),
        ],
        out_specs=[
            pl.BlockSpec((1, E, N), lambda b, nb: (b, 0, 0)),
            pl.BlockSpec((1, E, 128), lambda b, nb: (b, 0, 0)),
        ],
        out_shape=[
            jax.ShapeDtypeStruct((B, E, N), jnp.float32),
            jax.ShapeDtypeStruct((B, E, 128), jnp.int32),
        ],
        scratch_shapes=[pltpu.VMEM((E, N), jnp.float32)],
    )(x, gw)


def _select_sc(gates, meta):
    @functools.partial(
        pl.kernel,
        out_type=[
            jax.ShapeDtypeStruct((B, E, N), jnp.int32),
            jax.ShapeDtypeStruct((E, B, M), jnp.int32),
            jax.ShapeDtypeStruct((E, B, M), jnp.float32),
        ],
        mesh=_mesh(),
        compiler_params=_sc_params(),
        scratch_types=[
            pltpu.VMEM((N,), jnp.float32),
            pltpu.VMEM((N,), jnp.int32),
            pltpu.VMEM((M,), jnp.int32),
            pltpu.VMEM((M,), jnp.float32),
            pltpu.VMEM((128,), jnp.int32),
        ],
    )
    def k(gates_hbm, meta_hbm, claims_hbm, idx_hbm, gv_hbm,
          g_v, claims_v, idx_v, gv_v, meta_v):
        c = lax.axis_index("c")
        s = lax.axis_index("s")
        w = s * 2 + c
        b = w // E
        e = w % E
        pltpu.sync_copy(gates_hbm.at[b, e], g_v)
        pltpu.sync_copy(meta_hbm.at[b, e], meta_v)
        mv = meta_v[pl.ds(0, 16)]
        lane = lax.iota(jnp.int32, 16)
        thr = jnp.sum(jnp.where(lane == 0, mv, 0))
        need = jnp.sum(jnp.where(lane == 1, mv, 0))

        def chunk(ci, carry):
            cnt_eq, cnt_sel = carry
            gch = g_v[pl.ds(ci * 16, 16)]
            gint = plsc.bitcast(gch, jnp.int32)
            gt = gint > thr
            eq = gint == thr
            eqc = plsc.cumsum(eq.astype(jnp.int32))
            sel = jnp.logical_or(
                gt, jnp.logical_and(eq, (eqc + cnt_eq) <= need))
            seli = sel.astype(jnp.int32)
            selc = plsc.cumsum(seli)
            pos = selc + (cnt_sel - 1)
            slotv = (e * B + b) * M + pos
            claims_v[pl.ds(ci * 16, 16)] = jnp.where(sel, slotv + 1, 0)
            gidx = b * N + ci * 16 + lax.iota(jnp.int32, 16)
            plsc.store_scatter(idx_v, [pos], gidx, mask=sel)
            plsc.store_scatter(gv_v, [pos], gch, mask=sel)
            return (cnt_eq + jnp.sum(eq.astype(jnp.int32)),
                    cnt_sel + jnp.sum(seli))

        lax.fori_loop(0, N // 16, chunk, (jnp.int32(0), jnp.int32(0)))
        pltpu.sync_copy(claims_v, claims_hbm.at[b, e])
        pltpu.sync_copy(idx_v, idx_hbm.at[e, b])
        pltpu.sync_copy(gv_v, gv_hbm.at[e, b])

    return k(gates, meta)


def _winner_sc(claims):
    @functools.partial(
        pl.kernel,
        out_type=jax.ShapeDtypeStruct((B * N,), jnp.int32),
        mesh=_mesh(),
        compiler_params=_sc_params(),
        scratch_types=[
            pltpu.VMEM((E, M), jnp.int32),
            pltpu.VMEM((M,), jnp.int32),
        ],
    )
    def k(claims_hbm, slot_hbm, cv, slot_v):
        c = lax.axis_index("c")
        s = lax.axis_index("s")
        w = s * 2 + c
        b = w // E
        sl = w % E
        pltpu.sync_copy(claims_hbm.at[b, :, pl.ds(sl * M, M)], cv)

        def chunk(ci, _):
            mx = cv[0, pl.ds(ci * 16, 16)]
            for e_ in range(1, E):
                mx = jnp.maximum(mx, cv[e_, pl.ds(ci * 16, 16)])
            slot_v[pl.ds(ci * 16, 16)] = jnp.where(mx == 0, W * M, mx - 1)
            return 0

        lax.fori_loop(0, M // 16, chunk, 0)
        pltpu.sync_copy(slot_v, slot_hbm.at[pl.ds(w * M, M)])

    return k(claims)


def _gather_sc(table, idx_flat, nrows):
    rpw = nrows // 32
    nch = rpw // GW

    @functools.partial(
        pl.kernel,
        out_type=jax.ShapeDtypeStruct((nrows, D), jnp.float32),
        mesh=_mesh(),
        compiler_params=_sc_params(),
        scratch_types=[
            pltpu.VMEM((rpw,), jnp.int32),
            pltpu.VMEM((GW, D), jnp.float32),
            pltpu.VMEM((GW, D), jnp.float32),
            pltpu.SemaphoreType.DMA,
            pltpu.SemaphoreType.DMA,
        ],
    )
    def k(x_hbm, i_hbm, o_hbm, idx_v, buf0, buf1, sem0, sem1):
        c = lax.axis_index("c")
        s = lax.axis_index("s")
        w = s * 2 + c
        base = w * rpw
        pltpu.sync_copy(i_hbm.at[pl.ds(base, rpw)], idx_v)
        bufs = (buf0, buf1)
        sems = (sem0, sem1)
        pltpu.async_copy(x_hbm.at[idx_v.at[pl.ds(0, GW)]], buf0, sem0)
        for g in range(nch):
            buf, sem = bufs[g % 2], sems[g % 2]
            pltpu.make_async_copy(
                x_hbm.at[idx_v.at[pl.ds(g * GW, GW)]], buf, sem).wait()
            if g + 1 < nch:
                pltpu.async_copy(
                    x_hbm.at[idx_v.at[pl.ds((g + 1) * GW, GW)]],
                    bufs[(g + 1) % 2], sems[(g + 1) % 2])
            pltpu.sync_copy(buf, o_hbm.at[pl.ds(base + g * GW, GW)])

    return k(table, idx_flat)


def _k4_body(routed_ref, ex_ref, gv_ref, out_ref):
    eid = pl.program_id(0)

    @pl.when(eid < E)
    def _():
        rv = routed_ref[...].astype(jnp.bfloat16)
        ev = ex_ref[0].astype(jnp.bfloat16)
        acc = jnp.dot(rv, ev, preferred_element_type=jnp.float32)
        gch = gv_ref[0, 0]
        out_ref[...] = acc * gch[:, None]

    @pl.when(eid == E)
    def _():
        out_ref[...] = jnp.zeros_like(out_ref)


def _expert_tc(routed, experts, gv):
    gvr = gv.reshape(E, 1, B * M)
    return pl.pallas_call(
        _k4_body,
        grid=(E + 1, 4),
        in_specs=[
            pl.BlockSpec((B * M, D),
                         lambda e, j: (jnp.minimum(e, E - 1), 0)),
            pl.BlockSpec((1, D, D // 4),
                         lambda e, j: (jnp.minimum(e, E - 1), 0, j)),
            pl.BlockSpec((1, 1, B * M),
                         lambda e, j: (jnp.minimum(e, E - 1), 0, 0)),
        ],
        out_specs=pl.BlockSpec((B * M, D // 4), lambda e, j: (e, j)),
        out_shape=jax.ShapeDtypeStruct((W * M + NZ, D), jnp.float32),
    )(routed, experts, gvr)


def kernel(x, gate_weight, experts):
    gw = gate_weight.reshape(D, E)
    gates, meta = _route_tc(x, gw)
    claims, idx, gv = _select_sc(gates, meta)
    x_flat = x.reshape(B * N, D)
    routed = _gather_sc(x_flat, idx.reshape(W * M), W * M)
    slotmap = _winner_sc(claims)
    outs = _expert_tc(routed, experts, gv)
    final = _gather_sc(outs, slotmap, B * N)
    return final.reshape(B, N, D)
